# Initial kernel scaffold; baseline (speedup 1.0000x reference)
import jax
import jax.numpy as jnp
from jax.experimental import pallas as pl


def kernel(z, emb_w):
    raise NotImplementedError("write your pallas kernel here")


# speedup vs baseline: 6.9049x; 6.9049x over previous
import functools

import jax
import jax.numpy as jnp
from jax import lax
from jax.experimental import pallas as pl
from jax.experimental.pallas import tpu as pltpu
from jax.experimental.pallas import tpu_sc as plsc

N_CODES = 8192
DIM = 256
N_TOK = 8192
TOK_BLK = 1024
K_BLK = 1024


def _bf16_rne(x):
    u = lax.bitcast_convert_type(x, jnp.uint32)
    r = (u + jnp.uint32(0x7FFF) + ((u >> jnp.uint32(16)) & jnp.uint32(1)))
    r = r & jnp.uint32(0xFFFF0000)
    return lax.bitcast_convert_type(r, jnp.float32)


def _argmin_body(z_ref, e_ref, z2_ref, e2_ref, idx_ref):
    z = z_ref[...]
    z2 = z2_ref[0, 0, :]

    def body(c, carry):
        best_d, best_i = carry
        e = e_ref[pl.ds(c * K_BLK, K_BLK), :]
        e2 = e2_ref[0, pl.ds(c * K_BLK, K_BLK)]
        mm = lax.dot_general(z, e, (((1,), (1,)), ((), ())),
                             preferred_element_type=jnp.float32)
        d = (z2[:, None] + e2[None, :]) - 2.0 * mm
        m = jnp.min(d, axis=1)
        ii = lax.broadcasted_iota(jnp.int32, d.shape, 1)
        cand = jnp.where(d == m[:, None], ii, jnp.int32(2**30))
        a = jnp.min(cand, axis=1) + c * K_BLK
        upd = m < best_d
        return jnp.where(upd, m, best_d), jnp.where(upd, a, best_i)

    def half(c0, c1):
        init = (jnp.full((TOK_BLK,), jnp.inf, jnp.float32),
                jnp.zeros((TOK_BLK,), jnp.int32))
        return lax.fori_loop(c0, c1, body, init)

    nh = N_CODES // K_BLK // 2
    m_lo, a_lo = half(0, nh)
    m_hi, a_hi = half(nh, 2 * nh)
    use_hi = m_hi < _bf16_rne(m_lo)
    idx_ref[...] = jnp.where(use_hi, a_hi, a_lo)[None, None, :]


def _tc_argmin(z_flat, emb_w, z2, e2):
    return pl.pallas_call(
        _argmin_body,
        grid=(N_TOK // TOK_BLK,),
        in_specs=[
            pl.BlockSpec((TOK_BLK, DIM), lambda i: (i, 0)),
            pl.BlockSpec((N_CODES, DIM), lambda i: (0, 0)),
            pl.BlockSpec((1, 1, TOK_BLK), lambda i: (i, 0, 0)),
            pl.BlockSpec((1, N_CODES), lambda i: (0, 0)),
        ],
        out_specs=pl.BlockSpec((1, 1, TOK_BLK), lambda i: (i, 0, 0)),
        out_shape=jax.ShapeDtypeStruct((N_TOK // TOK_BLK, 1, TOK_BLK),
                                       jnp.int32),
    )(z_flat, emb_w, z2, e2)


_NW = 32
_ROWS_PER_W = N_TOK // _NW
_HIST_ROWS = N_CODES // 8
_HROWS_PER_S = _HIST_ROWS // 16


def _sc_body(emb_hbm, idx_hbm, zeros_hbm, pat_hbm, zq_hbm, cnt_hbm,
             idx_v, g_v, r_v, rows_v, src_v, cnt_v, hist, sem):
    cid = lax.axis_index("c")
    sid = lax.axis_index("s")
    wid = sid * 2 + cid
    pltpu.sync_copy(idx_hbm.at[pl.ds(wid * 2, 2)], idx_v)
    g0 = pltpu.async_copy(emb_hbm.at[idx_v.at[0]],
                          rows_v.at[pl.ds(0, 128)], sem)
    g1 = pltpu.async_copy(emb_hbm.at[idx_v.at[1]],
                          rows_v.at[pl.ds(128, 128)], sem)
    @pl.loop(0, 2)
    def _(j):
        @pl.loop(0, 8)
        def _(c):
            iv = idx_v[j, pl.ds(c * 16, 16)]
            g_v[j, pl.ds(c * 16, 16)] = iv & 7
            r_v[j, pl.ds(c * 16, 16)] = iv >> 3
    pltpu.sync_copy(pat_hbm.at[g_v.at[0]], src_v.at[pl.ds(0, 128)])
    pltpu.sync_copy(pat_hbm.at[g_v.at[1]], src_v.at[pl.ds(128, 128)])
    pltpu.sync_copy(zeros_hbm.at[pl.ds(sid * _HROWS_PER_S, _HROWS_PER_S)],
                    hist.at[pl.ds(sid * _HROWS_PER_S, _HROWS_PER_S)])
    plsc.subcore_barrier()
    pltpu.sync_copy(src_v.at[pl.ds(0, 128)], hist.at[r_v.at[0]], add=True)
    pltpu.sync_copy(src_v.at[pl.ds(128, 128)], hist.at[r_v.at[1]], add=True)
    plsc.subcore_barrier()
    pltpu.sync_copy(hist.at[pl.ds(sid * _HROWS_PER_S, _HROWS_PER_S)], cnt_v)
    pltpu.sync_copy(cnt_v,
                    cnt_hbm.at[pl.ds(cid * _HIST_ROWS + sid * _HROWS_PER_S,
                                     _HROWS_PER_S)])
    g0.wait()
    g1.wait()
    pltpu.sync_copy(rows_v, zq_hbm.at[pl.ds(wid * _ROWS_PER_W, _ROWS_PER_W)])


@jax.jit
def _sc_gather_count(emb_w, idx2d, zeros, pat):
    k = pl.kernel(
        _sc_body,
        mesh=plsc.VectorSubcoreMesh(core_axis_name="c", subcore_axis_name="s"),
        out_type=[
            jax.ShapeDtypeStruct((N_TOK, DIM), jnp.float32),
            jax.ShapeDtypeStruct((2 * _HIST_ROWS, 128), jnp.float32),
        ],
        scratch_types=[
            pltpu.VMEM((2, 128), jnp.int32),
            pltpu.VMEM((2, 128), jnp.int32),
            pltpu.VMEM((2, 128), jnp.int32),
            pltpu.VMEM((_ROWS_PER_W, DIM), jnp.float32),
            pltpu.VMEM((_ROWS_PER_W, 128), jnp.float32),
            pltpu.VMEM((_HROWS_PER_S, 128), jnp.float32),
            pltpu.VMEM_SHARED((_HIST_ROWS, 128), jnp.float32),
            pltpu.SemaphoreType.DMA,
        ],
    )
    return k(emb_w, idx2d, zeros, pat)


def _loss_body(z_ref, q_ref, cnt_ref, st_ref, loss_ref, perp_ref):
    z = z_ref[...]
    q = q_ref[...]
    dlt = q - z
    st_ref[...] = z + dlt
    sq = dlt * dlt
    loss_ref[0, 0] = 1.25 * (jnp.sum(sq) / jnp.float32(N_TOK * DIM))
    cnt = cnt_ref[0:_HIST_ROWS, :] + cnt_ref[_HIST_ROWS:2 * _HIST_ROWS, :]
    e_mean = cnt * jnp.float32(1.0 / N_TOK)
    ent = jnp.sum(e_mean * jnp.log(e_mean + 1e-10)) / jnp.float32(16.0)
    perp_ref[0, 0] = jnp.exp(-ent)


def _tc_loss(z_flat, zq_flat, cnt):
    return pl.pallas_call(
        _loss_body,
        in_specs=[
            pl.BlockSpec(memory_space=pltpu.VMEM),
            pl.BlockSpec(memory_space=pltpu.VMEM),
            pl.BlockSpec(memory_space=pltpu.VMEM),
        ],
        out_specs=[
            pl.BlockSpec(memory_space=pltpu.VMEM),
            pl.BlockSpec(memory_space=pltpu.SMEM),
            pl.BlockSpec(memory_space=pltpu.SMEM),
        ],
        out_shape=[
            jax.ShapeDtypeStruct((N_TOK, DIM), jnp.float32),
            jax.ShapeDtypeStruct((1, 1), jnp.float32),
            jax.ShapeDtypeStruct((1, 1), jnp.float32),
        ],
    )(z_flat, zq_flat, cnt)


def kernel(z, emb_w):
    B, E, L = z.shape
    zp = jnp.transpose(z, (0, 2, 1))
    z_flat = zp.reshape(N_TOK, DIM)
    z2 = jnp.sum(zp ** 2, axis=2).reshape(N_TOK // TOK_BLK, 1, TOK_BLK)
    e2 = jnp.sum(emb_w ** 2, axis=1).reshape(1, N_CODES)
    idx3 = _tc_argmin(z_flat, emb_w, z2, e2)
    idx_flat = idx3.reshape(N_TOK)
    pat = (lax.broadcasted_iota(jnp.int32, (8, 128), 1) // 16
           == lax.broadcasted_iota(jnp.int32, (8, 128), 0)).astype(jnp.float32)
    zq_flat, cnt = _sc_gather_count(
        emb_w, idx_flat.reshape(N_TOK // 128, 128),
        jnp.zeros((_HIST_ROWS, 128), jnp.float32), pat)
    st_flat, loss, perp = _tc_loss(z_flat, zq_flat, cnt)
    z_q = jnp.transpose(st_flat.reshape(B, L, E), (0, 2, 1))
    return (z_q, loss[0, 0], perp[0, 0], idx_flat[:, None])
